# Initial kernel scaffold; baseline (speedup 1.0000x reference)
import jax
import jax.numpy as jnp
from jax.experimental import pallas as pl


def kernel(X, adj_edge_index, adj_edge_attr, W1, b1, W2, b2, bn1_g, bn1_b, bn2_g, bn2_b, Wih1, Whh1, bih1, bhh1, Wih2, Whh2, bih2, bhh2, fc1W, fc1b, fc2W, fc2b):
    raise NotImplementedError("write your pallas kernel here")


# speedup vs baseline: 26.5509x; 26.5509x over previous
import functools

import jax
import jax.numpy as jnp
from jax import lax
from jax.experimental import pallas as pl
from jax.experimental.pallas import tpu as pltpu
from jax.experimental.pallas import tpu_sc as plsc

_N = 10000
_NP = 10240
_F = 128
_H = 64
_O = 32
_W = 4
_E = 160000

_NS = 16
_K = 80
_NCH = 128
_EP = _NS * _NCH * _K
_EPT = _NCH * _K
_NPT = _NP // _NS
_D = 2 * _H

_BN = 1000
_GRID = _N // _BN

_INV_SQRT1P = 1.0 / (1.0 + 1e-5) ** 0.5

_sc_mesh = plsc.VectorSubcoreMesh(core_axis_name="c", subcore_axis_name="s",
                                  num_cores=2, num_subcores=_NS)


@functools.partial(
    pl.kernel,
    out_type=[jax.ShapeDtypeStruct((_NP,), jnp.float32),
              jax.ShapeDtypeStruct((_NP,), jnp.float32)],
    mesh=_sc_mesh,
    scratch_types=[
        pltpu.VMEM_SHARED((_NP,), jnp.float32),
        pltpu.VMEM((_K,), jnp.int32),
        pltpu.VMEM((_K,), jnp.float32),
        pltpu.VMEM((_NPT,), jnp.float32),
    ],
)
def _deg_kernel(c_hbm, w_hbm, deg0_hbm, deg1_hbm, acc, cidx, wbuf, stg):
    c = lax.axis_index("c")
    s = lax.axis_index("s")
    zv = jnp.zeros((16,), jnp.float32)
    for i in range(_NPT // 16):
        stg[pl.ds(i * 16, 16)] = zv
    pltpu.sync_copy(stg, acc.at[pl.ds(s * _NPT, _NPT)])
    plsc.subcore_barrier()
    def chunk(j, carry):
        base = s * _EPT + (c * (_NCH // 2) + j) * _K
        pltpu.sync_copy(c_hbm.at[pl.ds(base, _K)], cidx)
        pltpu.sync_copy(w_hbm.at[pl.ds(base, _K)], wbuf)
        pltpu.sync_copy(wbuf, acc.at[cidx], add=True)
        return carry

    lax.fori_loop(0, _NCH // 2, chunk, 0)
    plsc.subcore_barrier()
    pltpu.sync_copy(acc.at[pl.ds(s * _NPT, _NPT)], stg)

    @pl.when(c == 0)
    def _():
        pltpu.sync_copy(stg, deg0_hbm.at[pl.ds(s * _NPT, _NPT)])

    @pl.when(c == 1)
    def _():
        pltpu.sync_copy(stg, deg1_hbm.at[pl.ds(s * _NPT, _NPT)])


@functools.partial(
    pl.kernel,
    out_type=[jax.ShapeDtypeStruct((_NP, _D), jnp.float32),
              jax.ShapeDtypeStruct((_NP, _D), jnp.float32)],
    mesh=_sc_mesh,
    scratch_types=[
        pltpu.VMEM_SHARED((_NP, _D), jnp.float32),
        pltpu.VMEM((_K,), jnp.int32),
        pltpu.VMEM((_K,), jnp.int32),
        pltpu.VMEM((_K,), jnp.float32),
        pltpu.VMEM((_K, _D), jnp.float32),
        pltpu.SemaphoreType.DMA,
    ],
)
def _spmm_kernel(hp0, hp1, r_hbm, c_hbm, w_hbm, agg0, agg1,
                 acc, ridx, cidx, wbuf, rows, sem):
    c = lax.axis_index("c")
    s = lax.axis_index("s")

    def run(hp, agg):
        for b in range(_NPT // _K):
            pltpu.sync_copy(hp.at[pl.ds(s * _NPT + b * _K, _K)], rows)
            pltpu.sync_copy(rows, acc.at[pl.ds(s * _NPT + b * _K, _K)])
        plsc.subcore_barrier()

        def chunk(j, jcarry):
            base = s * _EPT + j * _K
            pltpu.sync_copy(r_hbm.at[pl.ds(base, _K)], ridx)
            pltpu.sync_copy(c_hbm.at[pl.ds(base, _K)], cidx)
            pltpu.sync_copy(w_hbm.at[pl.ds(base, _K)], wbuf)
            pltpu.async_copy(hp.at[ridx], rows, sem).wait()

            for g in range(_K // 16):
                wv16 = wbuf[pl.ds(g * 16, 16)]
                for k in range(16):
                    wv = wv16.at[jnp.full((16,), k, jnp.int32)].get(
                        mode="promise_in_bounds")
                    e = g * 16 + k
                    for f in range(_D // 16):
                        rows[e, pl.ds(f * 16, 16)] = (
                            rows[e, pl.ds(f * 16, 16)] * wv)
            pltpu.sync_copy(rows, acc.at[cidx], add=True)
            return jcarry

        lax.fori_loop(0, _NCH, chunk, 0)
        plsc.subcore_barrier()
        for b in range(_NPT // _K):
            pltpu.sync_copy(acc.at[pl.ds(s * _NPT + b * _K, _K)], rows)
            pltpu.sync_copy(rows, agg.at[pl.ds(s * _NPT + b * _K, _K)])

    @pl.when(c == 0)
    def _():
        run(hp0, agg0)

    @pl.when(c == 1)
    def _():
        run(hp1, agg1)


def _tc_a_body(deg0, deg1, xT, W1, dis, hp0, hp1):
    deg = deg0[...] + deg1[...] + 1.0
    dv = lax.rsqrt(deg)
    dis[...] = dv
    for t in range(_W):
        h = jnp.dot(xT[t], W1[...], preferred_element_type=jnp.float32) * dv
        tgt = hp0 if t < 2 else hp1
        tgt[:, (t % 2) * _H:(t % 2 + 1) * _H] = h


def _tc_b_body(agg0, agg1, dis, b1, g1, be1, W2, h1out, hp0, hp1):
    dv = dis[...]
    s1 = g1[...] * _INV_SQRT1P
    for t in range(_W):
        src = agg0 if t < 2 else agg1
        a = src[:, (t % 2) * _H:(t % 2 + 1) * _H]
        h1 = jax.nn.relu(a * dv + b1[...]) * s1 + be1[...]
        h1out[t] = h1
        hp = jnp.dot(h1, W2[...], preferred_element_type=jnp.float32) * dv
        tgt = hp0 if t < 2 else hp1
        tgt[:, (t % 2) * _H:(t % 2 + 1) * _H] = hp


def _tc_c_body(agg0, agg1, dis, b2, g2, be2, h1in, skip,
               WihT1, WhhT1, bg1, WihT2, WhhT2, bg2,
               fc1WT, fc1b, fc2WT, fc2b, out):
    dv = dis[...]
    s2 = g2[...] * _INV_SQRT1P
    xs = []
    for t in range(_W):
        src = agg0 if t < 2 else agg1
        a = src[:, (t % 2) * _H:(t % 2 + 1) * _H]
        h2 = jax.nn.relu(a * dv + b2[...]) * s2 + be2[...]
        xs.append(jnp.concatenate([h1in[t], h2], axis=1))

    def cell(x_t, h, cc, WihT, WhhT, bg):
        gates = (jnp.dot(x_t, WihT[...], preferred_element_type=jnp.float32)
                 + jnp.dot(h, WhhT[...], preferred_element_type=jnp.float32)
                 + bg[...])
        i = jax.nn.sigmoid(gates[:, 0 * _H:1 * _H])
        f = jax.nn.sigmoid(gates[:, 1 * _H:2 * _H])
        g = jnp.tanh(gates[:, 2 * _H:3 * _H])
        o = jax.nn.sigmoid(gates[:, 3 * _H:4 * _H])
        cc = f * cc + i * g
        h = o * jnp.tanh(cc)
        return h, cc

    z0 = jnp.zeros((_BN, _H), jnp.float32)
    h, cc = z0, z0
    ys = []
    for t in range(_W):
        h, cc = cell(xs[t], h, cc, WihT1, WhhT1, bg1)
        ys.append(h)
    hn1 = h
    h, cc = z0, z0
    for t in range(_W):
        h, cc = cell(ys[t], h, cc, WihT2, WhhT2, bg2)
    hn2 = h

    z = (jnp.dot(hn1, fc1WT[0:_H, :], preferred_element_type=jnp.float32)
         + jnp.dot(hn2, fc1WT[_H:2 * _H, :], preferred_element_type=jnp.float32)
         + jnp.dot(skip[...], fc1WT[2 * _H:, :], preferred_element_type=jnp.float32)
         + fc1b[...])
    z = jax.nn.relu(z)
    z = jax.nn.relu(jnp.dot(z, fc2WT[...], preferred_element_type=jnp.float32)
                    + fc2b[...])
    out[...] = z


def _col(bn, d):
    return pl.BlockSpec((bn, d), lambda i: (i, 0))


def _full(shape):
    nd = len(shape)
    return pl.BlockSpec(shape, lambda i, _nd=nd: (0,) * _nd)


_tc_a = pl.pallas_call(
    _tc_a_body,
    grid=(_GRID,),
    in_specs=[_col(_BN, 1), _col(_BN, 1),
              pl.BlockSpec((_W, _BN, _F), lambda i: (0, i, 0)),
              _full((_F, _H))],
    out_specs=[_col(_BN, 1), _col(_BN, _D), _col(_BN, _D)],
    out_shape=[jax.ShapeDtypeStruct((_N, 1), jnp.float32),
               jax.ShapeDtypeStruct((_N, _D), jnp.float32),
               jax.ShapeDtypeStruct((_N, _D), jnp.float32)],
)

_tc_b = pl.pallas_call(
    _tc_b_body,
    grid=(_GRID,),
    in_specs=[_col(_BN, _D), _col(_BN, _D), _col(_BN, 1),
              _full((1, _H)), _full((1, _H)), _full((1, _H)),
              _full((_H, _H))],
    out_specs=[pl.BlockSpec((_W, _BN, _H), lambda i: (0, i, 0)),
               _col(_BN, _D), _col(_BN, _D)],
    out_shape=[jax.ShapeDtypeStruct((_W, _N, _H), jnp.float32),
               jax.ShapeDtypeStruct((_N, _D), jnp.float32),
               jax.ShapeDtypeStruct((_N, _D), jnp.float32)],
)

_tc_c = pl.pallas_call(
    _tc_c_body,
    grid=(_GRID,),
    in_specs=[_col(_BN, _D), _col(_BN, _D), _col(_BN, 1),
              _full((1, _H)), _full((1, _H)), _full((1, _H)),
              pl.BlockSpec((_W, _BN, _H), lambda i: (0, i, 0)),
              _col(_BN, _W * _F),
              _full((_D, 4 * _H)), _full((_H, 4 * _H)), _full((1, 4 * _H)),
              _full((_H, 4 * _H)), _full((_H, 4 * _H)), _full((1, 4 * _H)),
              _full((2 * _H + _W * _F, _H)), _full((1, _H)),
              _full((_H, _O)), _full((1, _O))],
    out_specs=[_col(_BN, _O)],
    out_shape=[jax.ShapeDtypeStruct((_N, _O), jnp.float32)],
)


def kernel(X, adj_edge_index, adj_edge_attr, W1, b1, W2, b2, bn1_g, bn1_b,
           bn2_g, bn2_b, Wih1, Whh1, bih1, bhh1, Wih2, Whh2, bih2, bhh2,
           fc1W, fc1b, fc2W, fc2b):
    fill = (jnp.arange(_EP - _E, dtype=jnp.int32) * 16) % _N
    r3 = jnp.concatenate([adj_edge_index[0], fill])
    c3 = jnp.concatenate([adj_edge_index[1], fill])
    w3 = jnp.pad(adj_edge_attr, (0, _EP - _E))

    deg0, deg1 = _deg_kernel(c3, w3)

    xT = jnp.transpose(X[0], (2, 0, 1))
    skipT = jnp.transpose(X[0], (0, 2, 1)).reshape(_N, _W * _F)

    dis, hp1_0, hp1_1 = _tc_a(deg0[:_N].reshape(_N, 1),
                              deg1[:_N].reshape(_N, 1), xT, W1)

    pad = ((0, _NP - _N), (0, 0))
    agg1_0, agg1_1 = _spmm_kernel(jnp.pad(hp1_0, pad), jnp.pad(hp1_1, pad),
                                  r3, c3, w3)

    h1, hp2_0, hp2_1 = _tc_b(agg1_0[:_N], agg1_1[:_N], dis,
                             b1.reshape(1, _H), bn1_g.reshape(1, _H),
                             bn1_b.reshape(1, _H), W2)

    agg2_0, agg2_1 = _spmm_kernel(jnp.pad(hp2_0, pad), jnp.pad(hp2_1, pad),
                                  r3, c3, w3)

    out, = _tc_c(agg2_0[:_N], agg2_1[:_N], dis,
                 b2.reshape(1, _H), bn2_g.reshape(1, _H), bn2_b.reshape(1, _H),
                 h1, skipT,
                 Wih1.T, Whh1.T, (bih1 + bhh1).reshape(1, 4 * _H),
                 Wih2.T, Whh2.T, (bih2 + bhh2).reshape(1, 4 * _H),
                 fc1W.T, fc1b.reshape(1, _H), fc2W.T, fc2b.reshape(1, _O))
    return out.reshape(1, _N, _O)
